# baseline (speedup 1.0000x reference)
import jax
import jax.numpy as jnp
from jax.experimental import pallas as pl
from jax.experimental.pallas import tpu as pltpu

N_IN = 16384
N_OUT = 128
B = 1024
KBLK = 2048
NB = N_IN // KBLK
NB2 = NB // 2


def _contrib(x, wvec, idxvec):
    onehot = jnp.where(
        idxvec[:, None] == jax.lax.broadcasted_iota(jnp.int32, (KBLK, N_OUT), 1),
        wvec[:, None],
        0.0,
    )
    return jnp.dot(x, onehot, preferred_element_type=jnp.float32)


def _spw_kernel(xa_ref, xb_ref, w_ref, idx_ref, gamma_ref, beta_ref, co_ref,
                out_ref, acc_ref):
    k = pl.program_id(0)
    ca = _contrib(xa_ref[...], w_ref[0, :KBLK], idx_ref[0, 0, :])
    cb = _contrib(xb_ref[...], w_ref[0, KBLK:], idx_ref[1, 0, :])
    contrib = ca + cb

    @pl.when(k == 0)
    def _init():
        acc_ref[...] = contrib

    @pl.when(k > 0)
    def _acc():
        acc_ref[...] += contrib

    @pl.when(k == NB2 - 1)
    def _finish():
        h = jnp.maximum(acc_ref[...], 0.0)
        mean = jnp.mean(h, axis=0, keepdims=True)
        d = h - mean
        var = jnp.mean(d * d, axis=0, keepdims=True)
        hn = d * jax.lax.rsqrt(var + 1e-5) * gamma_ref[...] + beta_ref[...]
        out_ref[...] = hn * jax.nn.sigmoid(co_ref[...])


@jax.jit
def kernel(x, weight, gamma, beta, co_weight, idx):
    idx3 = idx.astype(jnp.int32).reshape(NB, 1, KBLK)
    gamma2 = gamma.reshape(1, N_OUT)
    beta2 = beta.reshape(1, N_OUT)
    co2 = co_weight.reshape(1, N_OUT)
    return pl.pallas_call(
        _spw_kernel,
        grid=(NB2,),
        in_specs=[
            pl.BlockSpec((B, KBLK), lambda k: (0, 2 * k)),
            pl.BlockSpec((B, KBLK), lambda k: (0, 2 * k + 1)),
            pl.BlockSpec((1, 2 * KBLK), lambda k: (0, k)),
            pl.BlockSpec((2, 1, KBLK), lambda k: (k, 0, 0)),
            pl.BlockSpec((1, N_OUT), lambda k: (0, 0)),
            pl.BlockSpec((1, N_OUT), lambda k: (0, 0)),
            pl.BlockSpec((1, N_OUT), lambda k: (0, 0)),
        ],
        out_specs=pl.BlockSpec((B, N_OUT), lambda k: (0, 0)),
        out_shape=jax.ShapeDtypeStruct((B, N_OUT), jnp.float32),
        scratch_shapes=[pltpu.VMEM((B, N_OUT), jnp.float32)],
    )(x, x, weight, idx3, gamma2, beta2, co2)
